# baseline (speedup 1.0000x reference)
import functools

import numpy as np

import jax
import jax.numpy as jnp
from jax import lax
from jax.experimental import pallas as pl
from jax.experimental.pallas import tpu as pltpu
from jax.experimental.pallas import tpu_sc as plsc

B, S, H = 1024, 200, 128
V, P, T, E = 100000, 512, 2, 10
EPS = 1e-12
BS = B * S
NCOMBO = T * E * S
NPAD = 4096

NC, NS = 2, 16
NW = NC * NS
TOK_W = BS // NW
CH = 128
NIT = TOK_W // CH
CSTG = NPAD // NS


def _tc_combo(pos_s, type_table, event_table):
    def body(pos_ref, typ_ref, evt_ref, o_ref):
        p = pos_ref[...]
        for t in range(T):
            for e in range(E):
                row = typ_ref[t, :] + evt_ref[e, :]
                o_ref[pl.ds((t * E + e) * S, S), :] = (
                    p + lax.broadcast_in_dim(row, (S, H), (1,)))
        o_ref[pl.ds(NCOMBO, NPAD - NCOMBO), :] = jnp.zeros(
            (NPAD - NCOMBO, H), jnp.float32)

    return pl.pallas_call(
        body,
        out_shape=jax.ShapeDtypeStruct((NPAD, H), jnp.float32),
    )(pos_s, type_table, event_table)


def _tc_cidx(tt_ids, ev_ids):
    def body(tt_ref, ev_ref, o_ref):
        o_ref[...] = ((tt_ref[...] * E + ev_ref[...]) * S
                      + lax.broadcasted_iota(jnp.int32, (B, S), 1))

    return pl.pallas_call(
        body,
        out_shape=jax.ShapeDtypeStruct((B, S), jnp.int32),
    )(tt_ids, ev_ids)


def _sc_fused(word_table, ids_flat, cidx_flat, combo, gamma, beta):
    mesh = plsc.VectorSubcoreMesh(core_axis_name="c", subcore_axis_name="s")

    @functools.partial(
        pl.kernel,
        mesh=mesh,
        out_type=jax.ShapeDtypeStruct((BS, H), jnp.float32),
        scratch_types=[
            pltpu.VMEM_SHARED((NPAD, H), jnp.float32),
            pltpu.VMEM((CSTG // 2, H), jnp.float32),
            pltpu.VMEM((CH,), jnp.int32),
            pltpu.VMEM((CH,), jnp.int32),
            pltpu.VMEM((CH, H), jnp.float32),
            pltpu.VMEM((CH, H), jnp.float32),
            pltpu.VMEM((H,), jnp.float32),
            pltpu.VMEM((H,), jnp.float32),
            pltpu.SemaphoreType.DMA,
            pltpu.SemaphoreType.DMA,
        ],
    )
    def k(table_hbm, ids_hbm, cidx_hbm, combo_hbm, gamma_hbm, beta_hbm,
          out_hbm, combo_sp, stg_v, idx_v, cidx_v, rows_v, crow_v,
          g_v, b_v, sem_w, sem_c):
        cid = lax.axis_index("c")
        sid = lax.axis_index("s")
        wid = sid * NC + cid

        for p in range(2):
            r0 = sid * CSTG + p * (CSTG // 2)
            pltpu.sync_copy(combo_hbm.at[pl.ds(r0, CSTG // 2)], stg_v)
            pltpu.sync_copy(stg_v, combo_sp.at[pl.ds(r0, CSTG // 2)])
        pltpu.sync_copy(gamma_hbm, g_v)
        pltpu.sync_copy(beta_hbm, b_v)
        plsc.subcore_barrier()

        gs = [g_v[pl.ds(kk * 16, 16)] for kk in range(8)]
        bs = [b_v[pl.ds(kk * 16, 16)] for kk in range(8)]

        iota16 = lax.iota(jnp.int32, 16)
        bfly = [jnp.reshape(jnp.bitwise_xor(iota16, d), (16, 1))
                for d in (8, 4, 2, 1)]
        gdn = lax.GatherDimensionNumbers(
            offset_dims=(), collapsed_slice_dims=(0,), start_index_map=(0,))

        def xlane_sum(v):
            for idx in bfly:
                v = v + lax.gather(
                    v, idx, gdn, slice_sizes=(1,),
                    mode=lax.GatherScatterMode.PROMISE_IN_BOUNDS)
            return v

        def norm_row(r, carry):
            w = [rows_v[r, pl.ds(kk * 16, 16)] for kk in range(8)]
            c = [crow_v[r, pl.ds(kk * 16, 16)] for kk in range(8)]
            x = [w[kk] + c[kk] for kk in range(8)]
            ssum = x[0]
            ssq = x[0] * x[0]
            for kk in range(1, 8):
                ssum = ssum + x[kk]
                ssq = ssq + x[kk] * x[kk]
            mean = xlane_sum(ssum) * (1.0 / H)
            var = xlane_sum(ssq) * (1.0 / H) - mean * mean
            u = var + EPS
            m = u
            scale = bs[0] * 0.0 + 1.0
            for pw in (16, 8, 4, 2, 1):
                c = float(4.0 ** pw)
                cond = m >= c
                m = jnp.where(cond, m * (1.0 / c), m)
                scale = jnp.where(cond, scale * float(2.0 ** (-pw)), scale)
                cond2 = m < float(4.0 ** (-pw))
                m = jnp.where(cond2, m * c, m)
                scale = jnp.where(cond2, scale * float(2.0 ** pw), scale)
            y = 0.5 * (1.0 + 1.0 / m)
            for _ in range(4):
                y = y * (1.5 - 0.5 * m * y * y)
            y = y * scale
            for kk in range(8):
                crow_v[r, pl.ds(kk * 16, 16)] = (
                    (x[kk] - mean) * y * gs[kk] + bs[kk])
            return carry

        def body(j, carry):
            base = wid * TOK_W + j * CH
            pltpu.sync_copy(ids_hbm.at[pl.ds(base, CH)], idx_v)
            pltpu.sync_copy(cidx_hbm.at[pl.ds(base, CH)], cidx_v)
            cw = pltpu.async_copy(table_hbm.at[idx_v], rows_v, sem_w)
            cc = pltpu.async_copy(combo_sp.at[cidx_v], crow_v, sem_c)
            cw.wait()
            cc.wait()
            lax.fori_loop(0, CH, norm_row, 0)
            pltpu.sync_copy(crow_v, out_hbm.at[pl.ds(base, CH)])
            return carry

        lax.fori_loop(0, NIT, body, 0)

    return k(word_table, ids_flat, cidx_flat, combo, gamma, beta)


def kernel(input_ids, token_type_ids, event_type_ids, word_table, pos_table,
           type_table, event_table, gamma, beta):
    ids_flat = input_ids.reshape(BS).astype(jnp.int32)
    combo = _tc_combo(pos_table[:S], type_table, event_table)
    cidx_flat = _tc_cidx(token_type_ids, event_type_ids).reshape(BS)
    out = _sc_fused(word_table, ids_flat, cidx_flat, combo, gamma, beta)
    return out.reshape(B, S, H)


# speedup vs baseline: 3.1176x; 3.1176x over previous
import functools

import numpy as np

import jax
import jax.numpy as jnp
from jax import lax
from jax.experimental import pallas as pl
from jax.experimental.pallas import tpu as pltpu
from jax.experimental.pallas import tpu_sc as plsc

B, S, H = 1024, 200, 128
V, P, T, E = 100000, 512, 2, 10
EPS = 1e-12
BS = B * S
NCOMBO = T * E * S

NC, NS = 2, 16
NW = NC * NS
TOK_W = BS // NW
CH = 128
NIT = TOK_W // CH
NSTG = 10
CSTG = NCOMBO // NSTG


def _tc_combo(pos_s, type_table, event_table):
    def body(pos_ref, typ_ref, evt_ref, o_ref):
        p = pos_ref[...]
        for t in range(T):
            for e in range(E):
                row = typ_ref[t, :] + evt_ref[e, :]
                o_ref[pl.ds((t * E + e) * S, S), :] = (
                    p + lax.broadcast_in_dim(row, (S, H), (1,)))

    return pl.pallas_call(
        body,
        out_shape=jax.ShapeDtypeStruct((NCOMBO, H), jnp.float32),
    )(pos_s, type_table, event_table)


def _tc_cidx(tt_ids, ev_ids):
    def body(tt_ref, ev_ref, o_ref):
        o_ref[...] = ((tt_ref[...] * E + ev_ref[...]) * S
                      + lax.broadcasted_iota(jnp.int32, (B, S), 1))

    return pl.pallas_call(
        body,
        out_shape=jax.ShapeDtypeStruct((B, S), jnp.int32),
    )(tt_ids, ev_ids)


def _sc_fused(word_table, ids_flat, cidx_flat, combo, gamma, beta):
    mesh = plsc.VectorSubcoreMesh(core_axis_name="c", subcore_axis_name="s")

    @functools.partial(
        pl.kernel,
        mesh=mesh,
        out_type=jax.ShapeDtypeStruct((BS, H), jnp.float32),
        scratch_types=[
            pltpu.VMEM_SHARED((NCOMBO, H), jnp.float32),
            pltpu.VMEM((TOK_W,), jnp.int32),
            pltpu.VMEM((TOK_W,), jnp.int32),
            pltpu.VMEM((CH, H), jnp.float32),
            pltpu.VMEM((CH, H), jnp.float32),
            pltpu.VMEM((CH, H), jnp.float32),
            pltpu.VMEM((CH, H), jnp.float32),
            pltpu.VMEM((H,), jnp.float32),
            pltpu.VMEM((H,), jnp.float32),
            pltpu.SemaphoreType.DMA,
            pltpu.SemaphoreType.DMA,
            pltpu.SemaphoreType.DMA,
            pltpu.SemaphoreType.DMA,
            pltpu.SemaphoreType.DMA,
            pltpu.SemaphoreType.DMA,
        ],
    )
    def k(table_hbm, ids_hbm, cidx_hbm, combo_hbm, gamma_hbm, beta_hbm,
          out_hbm, combo_sp, idw_v, idc_v, rows0, rows1, obuf0, obuf1,
          g_v, b_v, sem_w0, sem_w1, sem_c0, sem_c1, sem_o0, sem_o1):
        cid = lax.axis_index("c")
        sid = lax.axis_index("s")
        wid = sid * NC + cid
        tok0 = wid * TOK_W
        slots = [
            (rows0, obuf0, sem_w0, sem_c0, sem_o0),
            (rows1, obuf1, sem_w1, sem_c1, sem_o1),
        ]

        @pl.when(sid < NSTG)
        def _():
            r0 = sid * CSTG
            for off, sz in ((0, 128), (128, 128), (256, 128), (384, 16)):
                pltpu.sync_copy(combo_hbm.at[pl.ds(r0 + off, sz)],
                                rows0.at[pl.ds(0, sz)])
                pltpu.sync_copy(rows0.at[pl.ds(0, sz)],
                                combo_sp.at[pl.ds(r0 + off, sz)])
        pltpu.sync_copy(ids_hbm.at[pl.ds(tok0, TOK_W)], idw_v)
        pltpu.sync_copy(cidx_hbm.at[pl.ds(tok0, TOK_W)], idc_v)
        pltpu.sync_copy(gamma_hbm, g_v)
        pltpu.sync_copy(beta_hbm, b_v)
        plsc.subcore_barrier()

        gs = [g_v[pl.ds(kk * 16, 16)] for kk in range(8)]
        bs = [b_v[pl.ds(kk * 16, 16)] for kk in range(8)]

        iota16 = lax.iota(jnp.int32, 16)
        bfly = [jnp.reshape(jnp.bitwise_xor(iota16, d), (16, 1))
                for d in (8, 4, 2, 1)]
        gdn = lax.GatherDimensionNumbers(
            offset_dims=(), collapsed_slice_dims=(0,), start_index_map=(0,))

        def xlane_sum(v):
            for idx in bfly:
                v = v + lax.gather(
                    v, idx, gdn, slice_sizes=(1,),
                    mode=lax.GatherScatterMode.PROMISE_IN_BOUNDS)
            return v

        lane_eq = [iota16 == q for q in range(16)]
        bq = [jnp.reshape(jnp.bitwise_xor(iota16, iota16) + q, (16, 1))
              for q in range(16)]

        def lane_bcast(v, q):
            return lax.gather(v, bq[q], gdn, slice_sizes=(1,),
                              mode=lax.GatherScatterMode.PROMISE_IN_BOUNDS)

        def rsqrt16(u):
            m = u
            scale = u * 0.0 + 1.0
            for pw in (16, 8, 4, 2, 1):
                c = float(4.0 ** pw)
                cond = m >= c
                m = jnp.where(cond, m * (1.0 / c), m)
                scale = jnp.where(cond, scale * float(2.0 ** (-pw)), scale)
                cond2 = m < float(4.0 ** (-pw))
                m = jnp.where(cond2, m * c, m)
                scale = jnp.where(cond2, scale * float(2.0 ** pw), scale)
            y = 0.5 * (1.0 + 1.0 / m)
            for _ in range(4):
                y = y * (1.5 - 0.5 * m * y * y)
            return y * scale

        def make_norm(rows_v, obuf_v):
            def norm_group(g, carry):
                r0g = g * 16
                mvec = iota16 * 0.0
                vvec = iota16 * 0.0
                for q in range(16):
                    r = r0g + q
                    x = [rows_v[r, pl.ds(kk * 16, 16)] for kk in range(8)]
                    ssum = x[0]
                    ssq = x[0] * x[0]
                    for kk in range(1, 8):
                        ssum = ssum + x[kk]
                        ssq = ssq + x[kk] * x[kk]
                    mean = xlane_sum(ssum) * (1.0 / H)
                    var = xlane_sum(ssq) * (1.0 / H) - mean * mean
                    mvec = jnp.where(lane_eq[q], mean, mvec)
                    vvec = jnp.where(lane_eq[q], var, vvec)
                rstd = rsqrt16(vvec + EPS)
                for q in range(16):
                    r = r0g + q
                    mq = lane_bcast(mvec, q)
                    yq = lane_bcast(rstd, q)
                    for kk in range(8):
                        obuf_v[r, pl.ds(kk * 16, 16)] = (
                            (rows_v[r, pl.ds(kk * 16, 16)] - mq) * yq
                            * gs[kk] + bs[kk])
                return carry

            lax.fori_loop(0, CH // 16, norm_group, 0)

        def word_gather(j, rows_v, sw):
            return pltpu.make_async_copy(
                table_hbm.at[idw_v.at[pl.ds(j * CH, CH)]], rows_v, sw)

        def issue_word(j, rows_v, sw):
            pltpu.async_copy(
                table_hbm.at[idw_v.at[pl.ds(j * CH, CH)]], rows_v, sw)

        def combo_add(j, rows_v, sc2):
            return pltpu.make_async_copy(
                combo_sp.at[idc_v.at[pl.ds(j * CH, CH)]], rows_v, sc2)

        def issue_combo(j, rows_v, sc2):
            pltpu.async_copy(
                combo_sp.at[idc_v.at[pl.ds(j * CH, CH)]], rows_v, sc2,
                add=True)

        NITO = NIT // 2
        issue_word(0, rows0, sem_w0)
        issue_word(1, rows1, sem_w1)
        word_gather(0, rows0, sem_w0).wait()
        issue_combo(0, rows0, sem_c0)

        def outer(jo, carry):
            for b2 in range(2):
                rows_v, obuf_v, sw, sc2, so = slots[b2]
                nrows_v, _, nsw, nsc, _ = slots[1 - b2]
                j = 2 * jo + b2
                base = tok0 + j * CH

                combo_add(j, rows_v, sc2).wait()

                @pl.when(jo > 0)
                def _():
                    pltpu.make_async_copy(
                        obuf_v, out_hbm.at[pl.ds(base - 2 * CH, CH)],
                        so).wait()

                make_norm(rows_v, obuf_v)
                pltpu.async_copy(obuf_v, out_hbm.at[pl.ds(base, CH)], so)

                @pl.when(jo < NITO - 1)
                def _():
                    issue_word(j + 2, rows_v, sw)

                if b2 == 0:
                    word_gather(j + 1, nrows_v, nsw).wait()
                    issue_combo(j + 1, nrows_v, nsc)
                else:
                    @pl.when(jo < NITO - 1)
                    def _():
                        word_gather(j + 1, nrows_v, nsw).wait()
                        issue_combo(j + 1, nrows_v, nsc)
            return carry

        lax.fori_loop(0, NITO, outer, 0)
        for b2 in range(2):
            rows_v, obuf_v, sw, sc2, so = slots[b2]
            last = tok0 + (NIT - 2 + b2) * CH
            pltpu.make_async_copy(
                obuf_v, out_hbm.at[pl.ds(last, CH)], so).wait()

    return k(word_table, ids_flat, cidx_flat, combo, gamma, beta)


def kernel(input_ids, token_type_ids, event_type_ids, word_table, pos_table,
           type_table, event_table, gamma, beta):
    ids_flat = input_ids.reshape(BS).astype(jnp.int32)
    combo = _tc_combo(pos_table[:S], type_table, event_table)
    cidx_flat = _tc_cidx(token_type_ids, event_type_ids).reshape(BS)
    out = _sc_fused(word_table, ids_flat, cidx_flat, combo, gamma, beta)
    return out.reshape(B, S, H)


# speedup vs baseline: 3.1290x; 1.0036x over previous
import functools

import numpy as np

import jax
import jax.numpy as jnp
from jax import lax
from jax.experimental import pallas as pl
from jax.experimental.pallas import tpu as pltpu
from jax.experimental.pallas import tpu_sc as plsc

B, S, H = 1024, 200, 128
V, P, T, E = 100000, 512, 2, 10
EPS = 1e-12
BS = B * S
NCOMBO = T * E * S

NC, NS = 2, 16
NW = NC * NS
TOK_W = BS // NW
CH = 128
NIT = TOK_W // CH
NSTG = 10
CSTG = NCOMBO // NSTG


def _tc_combo(pos_s, type_table, event_table):
    def body(pos_ref, typ_ref, evt_ref, o_ref):
        p = pos_ref[...]
        for t in range(T):
            for e in range(E):
                row = typ_ref[t, :] + evt_ref[e, :]
                o_ref[pl.ds((t * E + e) * S, S), :] = (
                    p + lax.broadcast_in_dim(row, (S, H), (1,)))

    return pl.pallas_call(
        body,
        out_shape=jax.ShapeDtypeStruct((NCOMBO, H), jnp.float32),
    )(pos_s, type_table, event_table)


def _tc_cidx(tt_ids, ev_ids):
    def body(tt_ref, ev_ref, o_ref):
        o_ref[...] = ((tt_ref[...] * E + ev_ref[...]) * S
                      + lax.broadcasted_iota(jnp.int32, (B, S), 1))

    return pl.pallas_call(
        body,
        out_shape=jax.ShapeDtypeStruct((B, S), jnp.int32),
    )(tt_ids, ev_ids)


def _sc_fused(word_table, ids_flat, cidx_flat, combo, gamma, beta):
    mesh = plsc.VectorSubcoreMesh(core_axis_name="c", subcore_axis_name="s")

    @functools.partial(
        pl.kernel,
        mesh=mesh,
        out_type=jax.ShapeDtypeStruct((BS, H), jnp.float32),
        scratch_types=[
            pltpu.VMEM_SHARED((NCOMBO, H), jnp.float32),
            pltpu.VMEM((TOK_W,), jnp.int32),
            pltpu.VMEM((TOK_W,), jnp.int32),
            pltpu.VMEM((CH, H), jnp.float32),
            pltpu.VMEM((CH, H), jnp.float32),
            pltpu.VMEM((CH, H), jnp.float32),
            pltpu.VMEM((CH, H), jnp.float32),
            pltpu.VMEM((H,), jnp.float32),
            pltpu.VMEM((H,), jnp.float32),
            pltpu.SemaphoreType.DMA,
            pltpu.SemaphoreType.DMA,
            pltpu.SemaphoreType.DMA,
            pltpu.SemaphoreType.DMA,
            pltpu.SemaphoreType.DMA,
            pltpu.SemaphoreType.DMA,
        ],
    )
    def k(table_hbm, ids_hbm, cidx_hbm, combo_hbm, gamma_hbm, beta_hbm,
          out_hbm, combo_sp, idw_v, idc_v, rows0, rows1, obuf0, obuf1,
          g_v, b_v, sem_w0, sem_w1, sem_c0, sem_c1, sem_o0, sem_o1):
        cid = lax.axis_index("c")
        sid = lax.axis_index("s")
        wid = sid * NC + cid
        tok0 = wid * TOK_W
        slots = [
            (rows0, obuf0, sem_w0, sem_c0, sem_o0),
            (rows1, obuf1, sem_w1, sem_c1, sem_o1),
        ]

        @pl.when(sid < NSTG)
        def _():
            r0 = sid * CSTG
            for off, sz in ((0, 128), (128, 128), (256, 128), (384, 16)):
                pltpu.sync_copy(combo_hbm.at[pl.ds(r0 + off, sz)],
                                rows0.at[pl.ds(0, sz)])
                pltpu.sync_copy(rows0.at[pl.ds(0, sz)],
                                combo_sp.at[pl.ds(r0 + off, sz)])
        pltpu.sync_copy(ids_hbm.at[pl.ds(tok0, TOK_W)], idw_v)
        pltpu.sync_copy(cidx_hbm.at[pl.ds(tok0, TOK_W)], idc_v)
        pltpu.sync_copy(gamma_hbm, g_v)
        pltpu.sync_copy(beta_hbm, b_v)
        plsc.subcore_barrier()

        gs = [g_v[pl.ds(kk * 16, 16)] for kk in range(8)]
        bs = [b_v[pl.ds(kk * 16, 16)] for kk in range(8)]

        iota16 = lax.iota(jnp.int32, 16)
        bfly = [jnp.reshape(jnp.bitwise_xor(iota16, d), (16, 1))
                for d in (8, 4, 2, 1)]
        gdn = lax.GatherDimensionNumbers(
            offset_dims=(), collapsed_slice_dims=(0,), start_index_map=(0,))

        def xlane_sum(v):
            for idx in bfly:
                v = v + lax.gather(
                    v, idx, gdn, slice_sizes=(1,),
                    mode=lax.GatherScatterMode.PROMISE_IN_BOUNDS)
            return v

        lane_eq = [iota16 == q for q in range(16)]
        bq = [jnp.reshape(jnp.bitwise_xor(iota16, iota16) + q, (16, 1))
              for q in range(16)]

        def lane_bcast(v, q):
            return lax.gather(v, bq[q], gdn, slice_sizes=(1,),
                              mode=lax.GatherScatterMode.PROMISE_IN_BOUNDS)

        def rsqrt16(u):
            m = u
            scale = u * 0.0 + 1.0
            for pw in (16, 8, 4, 2, 1):
                c = float(4.0 ** pw)
                cond = m >= c
                m = jnp.where(cond, m * (1.0 / c), m)
                scale = jnp.where(cond, scale * float(2.0 ** (-pw)), scale)
                cond2 = m < float(4.0 ** (-pw))
                m = jnp.where(cond2, m * c, m)
                scale = jnp.where(cond2, scale * float(2.0 ** pw), scale)
            y = 0.5 * (1.0 + 1.0 / m)
            for _ in range(4):
                y = y * (1.5 - 0.5 * m * y * y)
            return y * scale

        def make_norm(rows_v, obuf_v, lo, hi):
            def norm_group(g, carry):
                r0g = g * 16
                mvec = iota16 * 0.0
                vvec = iota16 * 0.0
                for q in range(16):
                    r = r0g + q
                    x = [rows_v[r, pl.ds(kk * 16, 16)] for kk in range(8)]
                    ssum = x[0]
                    ssq = x[0] * x[0]
                    for kk in range(1, 8):
                        ssum = ssum + x[kk]
                        ssq = ssq + x[kk] * x[kk]
                    mean = xlane_sum(ssum) * (1.0 / H)
                    var = xlane_sum(ssq) * (1.0 / H) - mean * mean
                    mvec = jnp.where(lane_eq[q], mean, mvec)
                    vvec = jnp.where(lane_eq[q], var, vvec)
                rstd = rsqrt16(vvec + EPS)
                for q in range(16):
                    r = r0g + q
                    mq = lane_bcast(mvec, q)
                    yq = lane_bcast(rstd, q)
                    for kk in range(8):
                        obuf_v[r, pl.ds(kk * 16, 16)] = (
                            (rows_v[r, pl.ds(kk * 16, 16)] - mq) * yq
                            * gs[kk] + bs[kk])
                return carry

            lax.fori_loop(lo, hi, norm_group, 0)

        def word_gather(j, rows_v, sw):
            return pltpu.make_async_copy(
                table_hbm.at[idw_v.at[pl.ds(j * CH, CH)]], rows_v, sw)

        def issue_word(j, rows_v, sw):
            pltpu.async_copy(
                table_hbm.at[idw_v.at[pl.ds(j * CH, CH)]], rows_v, sw)

        def combo_add(j, rows_v, sc2):
            return pltpu.make_async_copy(
                combo_sp.at[idc_v.at[pl.ds(j * CH, CH)]], rows_v, sc2)

        def issue_combo(j, rows_v, sc2):
            pltpu.async_copy(
                combo_sp.at[idc_v.at[pl.ds(j * CH, CH)]], rows_v, sc2,
                add=True)

        NITO = NIT // 2
        issue_word(0, rows0, sem_w0)
        issue_word(1, rows1, sem_w1)
        word_gather(0, rows0, sem_w0).wait()
        issue_combo(0, rows0, sem_c0)

        def outer(jo, carry):
            for b2 in range(2):
                rows_v, obuf_v, sw, sc2, so = slots[b2]
                nrows_v, _, nsw, nsc, _ = slots[1 - b2]
                j = 2 * jo + b2
                base = tok0 + j * CH

                combo_add(j, rows_v, sc2).wait()

                @pl.when(jo > 0)
                def _():
                    pltpu.make_async_copy(
                        obuf_v, out_hbm.at[pl.ds(base - 2 * CH, CH)],
                        so).wait()

                make_norm(rows_v, obuf_v, 0, CH // 32)
                if b2 == 0:
                    word_gather(j + 1, nrows_v, nsw).wait()
                    issue_combo(j + 1, nrows_v, nsc)
                else:
                    @pl.when(jo < NITO - 1)
                    def _():
                        word_gather(j + 1, nrows_v, nsw).wait()
                        issue_combo(j + 1, nrows_v, nsc)
                make_norm(rows_v, obuf_v, CH // 32, CH // 16)

                pltpu.async_copy(obuf_v, out_hbm.at[pl.ds(base, CH)], so)

                @pl.when(jo < NITO - 1)
                def _():
                    issue_word(j + 2, rows_v, sw)
            return carry

        lax.fori_loop(0, NITO, outer, 0)
        for b2 in range(2):
            rows_v, obuf_v, sw, sc2, so = slots[b2]
            last = tok0 + (NIT - 2 + b2) * CH
            pltpu.make_async_copy(
                obuf_v, out_hbm.at[pl.ds(last, CH)], so).wait()

    return k(word_table, ids_flat, cidx_flat, combo, gamma, beta)


def kernel(input_ids, token_type_ids, event_type_ids, word_table, pos_table,
           type_table, event_table, gamma, beta):
    ids_flat = input_ids.reshape(BS).astype(jnp.int32)
    combo = _tc_combo(pos_table[:S], type_table, event_table)
    cidx_flat = _tc_cidx(token_type_ids, event_type_ids).reshape(BS)
    out = _sc_fused(word_table, ids_flat, cidx_flat, combo, gamma, beta)
    return out.reshape(B, S, H)


# speedup vs baseline: 3.1365x; 1.0024x over previous
import functools

import jax
import jax.numpy as jnp
from jax import lax
from jax.experimental import pallas as pl
from jax.experimental.pallas import tpu as pltpu
from jax.experimental.pallas import tpu_sc as plsc

B, S, H = 1024, 200, 128
V, P, T, E = 100000, 512, 2, 10
EPS = 1e-12
BS = B * S
NCOMBO = T * E * S

NC, NS = 2, 16
NW = NC * NS
TOK_W = BS // NW
CH = 128
NIT = TOK_W // CH
NSTG = 10
CSTG = NCOMBO // NSTG


def _tc_combo(pos_s, type_table, event_table):
    def body(pos_ref, typ_ref, evt_ref, o_ref):
        p = pos_ref[...]
        for t in range(T):
            for e in range(E):
                row = typ_ref[t, :] + evt_ref[e, :]
                o_ref[pl.ds((t * E + e) * S, S), :] = (
                    p + lax.broadcast_in_dim(row, (S, H), (1,)))

    return pl.pallas_call(
        body,
        out_shape=jax.ShapeDtypeStruct((NCOMBO, H), jnp.float32),
    )(pos_s, type_table, event_table)


def _tc_cidx(tt_ids, ev_ids):
    def body(tt_ref, ev_ref, o_ref):
        o_ref[...] = ((tt_ref[...] * E + ev_ref[...]) * S
                      + lax.broadcasted_iota(jnp.int32, (B, S), 1))

    return pl.pallas_call(
        body,
        out_shape=jax.ShapeDtypeStruct((B, S), jnp.int32),
    )(tt_ids, ev_ids)


def _sc_fused(word_table, ids_flat, cidx_flat, combo, gamma, beta):
    mesh = plsc.VectorSubcoreMesh(core_axis_name="c", subcore_axis_name="s")

    @functools.partial(
        pl.kernel,
        mesh=mesh,
        out_type=jax.ShapeDtypeStruct((BS, H), jnp.float32),
        scratch_types=[
            pltpu.VMEM_SHARED((NCOMBO, H), jnp.float32),
            pltpu.VMEM((TOK_W,), jnp.int32),
            pltpu.VMEM((TOK_W,), jnp.int32),
            pltpu.VMEM((CH, H), jnp.float32),
            pltpu.VMEM((CH, H), jnp.float32),
            pltpu.VMEM((CH, H), jnp.float32),
            pltpu.VMEM((CH, H), jnp.float32),
            pltpu.VMEM((H,), jnp.float32),
            pltpu.VMEM((H,), jnp.float32),
            pltpu.SemaphoreType.DMA,
            pltpu.SemaphoreType.DMA,
            pltpu.SemaphoreType.DMA,
            pltpu.SemaphoreType.DMA,
            pltpu.SemaphoreType.DMA,
            pltpu.SemaphoreType.DMA,
        ],
    )
    def k(table_hbm, ids_hbm, cidx_hbm, combo_hbm, gamma_hbm, beta_hbm,
          out_hbm, combo_sp, idw_v, idc_v, rows0, rows1, obuf0, obuf1,
          g_v, b_v, sem_w0, sem_w1, sem_c0, sem_c1, sem_o0, sem_o1):
        cid = lax.axis_index("c")
        sid = lax.axis_index("s")
        wid = sid * NC + cid
        tok0 = wid * TOK_W
        slots = [
            (rows0, obuf0, sem_w0, sem_c0, sem_o0),
            (rows1, obuf1, sem_w1, sem_c1, sem_o1),
        ]

        @pl.when(sid < NSTG)
        def _():
            r0 = sid * CSTG
            for off, sz in ((0, 128), (128, 128), (256, 128), (384, 16)):
                pltpu.sync_copy(combo_hbm.at[pl.ds(r0 + off, sz)],
                                rows0.at[pl.ds(0, sz)])
                pltpu.sync_copy(rows0.at[pl.ds(0, sz)],
                                combo_sp.at[pl.ds(r0 + off, sz)])
        pltpu.sync_copy(ids_hbm.at[pl.ds(tok0, TOK_W)], idw_v)
        pltpu.sync_copy(cidx_hbm.at[pl.ds(tok0, TOK_W)], idc_v)
        pltpu.sync_copy(gamma_hbm, g_v)
        pltpu.sync_copy(beta_hbm, b_v)
        plsc.subcore_barrier()

        gs = [g_v[pl.ds(kk * 16, 16)] for kk in range(8)]
        bs = [b_v[pl.ds(kk * 16, 16)] for kk in range(8)]

        iota16 = lax.iota(jnp.int32, 16)
        bfly = [jnp.reshape(jnp.bitwise_xor(iota16, d), (16, 1))
                for d in (8, 4, 2, 1)]
        gdn = lax.GatherDimensionNumbers(
            offset_dims=(), collapsed_slice_dims=(0,), start_index_map=(0,))

        def xlane_sum(v):
            for idx in bfly:
                v = v + lax.gather(
                    v, idx, gdn, slice_sizes=(1,),
                    mode=lax.GatherScatterMode.PROMISE_IN_BOUNDS)
            return v

        lane_eq = [iota16 == q for q in range(16)]
        bq = [jnp.reshape(jnp.bitwise_xor(iota16, iota16) + q, (16, 1))
              for q in range(16)]

        def lane_bcast(v, q):
            return lax.gather(v, bq[q], gdn, slice_sizes=(1,),
                              mode=lax.GatherScatterMode.PROMISE_IN_BOUNDS)

        def rsqrt16(u):
            m = u
            scale = u * 0.0 + 1.0
            for pw in (16, 8, 4, 2, 1):
                c = float(4.0 ** pw)
                cond = m >= c
                m = jnp.where(cond, m * (1.0 / c), m)
                scale = jnp.where(cond, scale * float(2.0 ** (-pw)), scale)
                cond2 = m < float(4.0 ** (-pw))
                m = jnp.where(cond2, m * c, m)
                scale = jnp.where(cond2, scale * float(2.0 ** pw), scale)
            y = 0.5 * (1.0 + 1.0 / m)
            for _ in range(4):
                y = y * (1.5 - 0.5 * m * y * y)
            return y * scale

        def make_norm(rows_v, obuf_v, lo, hi):
            def norm_group(g, carry):
                r0g = g * 16
                mvec = iota16 * 0.0
                vvec = iota16 * 0.0
                for q in range(16):
                    r = r0g + q
                    x = [rows_v[r, pl.ds(kk * 16, 16)] for kk in range(8)]
                    ssum = x[0]
                    ssq = x[0] * x[0]
                    for kk in range(1, 8):
                        ssum = ssum + x[kk]
                        ssq = ssq + x[kk] * x[kk]
                    mean = xlane_sum(ssum) * (1.0 / H)
                    var = xlane_sum(ssq) * (1.0 / H) - mean * mean
                    mvec = jnp.where(lane_eq[q], mean, mvec)
                    vvec = jnp.where(lane_eq[q], var, vvec)
                rstd = rsqrt16(vvec + EPS)
                for q in range(16):
                    r = r0g + q
                    mq = lane_bcast(mvec, q)
                    yq = lane_bcast(rstd, q)
                    for kk in range(8):
                        obuf_v[r, pl.ds(kk * 16, 16)] = (
                            (rows_v[r, pl.ds(kk * 16, 16)] - mq) * yq
                            * gs[kk] + bs[kk])
                return carry

            lax.fori_loop(lo, hi, norm_group, 0)

        def word_gather(j, rows_v, sw):
            return pltpu.make_async_copy(
                table_hbm.at[idw_v.at[pl.ds(j * CH, CH)]], rows_v, sw)

        def issue_word(j, rows_v, sw):
            pltpu.async_copy(
                table_hbm.at[idw_v.at[pl.ds(j * CH, CH)]], rows_v, sw)

        def combo_add(j, rows_v, sc2):
            return pltpu.make_async_copy(
                combo_sp.at[idc_v.at[pl.ds(j * CH, CH)]], rows_v, sc2)

        def issue_combo(j, rows_v, sc2):
            pltpu.async_copy(
                combo_sp.at[idc_v.at[pl.ds(j * CH, CH)]], rows_v, sc2,
                add=True)

        NITO = NIT // 2
        issue_word(0, rows0, sem_w0)
        issue_word(1, rows1, sem_w1)
        word_gather(0, rows0, sem_w0).wait()
        issue_combo(0, rows0, sem_c0)

        def outer(jo, carry):
            for b2 in range(2):
                rows_v, obuf_v, sw, sc2, so = slots[b2]
                nrows_v, _, nsw, nsc, _ = slots[1 - b2]
                j = 2 * jo + b2
                base = tok0 + j * CH

                combo_add(j, rows_v, sc2).wait()

                @pl.when(jo > 0)
                def _():
                    pltpu.make_async_copy(
                        obuf_v, out_hbm.at[pl.ds(base - 2 * CH, CH)],
                        so).wait()

                make_norm(rows_v, obuf_v, 0, CH // 32)
                if b2 == 0:
                    word_gather(j + 1, nrows_v, nsw).wait()
                    issue_combo(j + 1, nrows_v, nsc)
                else:
                    @pl.when(jo < NITO - 1)
                    def _():
                        word_gather(j + 1, nrows_v, nsw).wait()
                        issue_combo(j + 1, nrows_v, nsc)
                make_norm(rows_v, obuf_v, CH // 32, CH // 16)

                pltpu.async_copy(obuf_v, out_hbm.at[pl.ds(base, CH)], so)

                @pl.when(jo < NITO - 1)
                def _():
                    issue_word(j + 2, rows_v, sw)
            return carry

        lax.fori_loop(0, NITO, outer, 0)
        for b2 in range(2):
            rows_v, obuf_v, sw, sc2, so = slots[b2]
            last = tok0 + (NIT - 2 + b2) * CH
            pltpu.make_async_copy(
                obuf_v, out_hbm.at[pl.ds(last, CH)], so).wait()

    return k(word_table, ids_flat, cidx_flat, combo, gamma, beta)


def kernel(input_ids, token_type_ids, event_type_ids, word_table, pos_table,
           type_table, event_table, gamma, beta):
    ids_flat = input_ids.reshape(BS).astype(jnp.int32)
    combo = _tc_combo(pos_table[:S], type_table, event_table)
    cidx_flat = _tc_cidx(token_type_ids, event_type_ids).reshape(BS)
    out = _sc_fused(word_table, ids_flat, cidx_flat, combo, gamma, beta)
    return out.reshape(B, S, H)
